# baseline (speedup 1.0000x reference)
import functools
import math

import jax
import jax.numpy as jnp
from jax import lax
from jax.experimental import pallas as pl
from jax.experimental.pallas import tpu as pltpu
from jax.experimental.pallas import tpu_sc as plsc

_SCALE = 64.0
_COS_M = math.cos(0.5)
_SIN_M = math.sin(0.5)

_NC = 2
_NS = 16
_NW = _NC * _NS


def _rsqrt_newton(x):
    i = plsc.bitcast(x, jnp.int32)
    y = plsc.bitcast(jnp.int32(0x5F3759DF) - (i >> 1), jnp.float32)
    for _ in range(4):
        y = y * (1.5 - 0.5 * x * y * y)
    return y


def _sc_body(logits_hbm, labels_hbm, out_hbm, row_v, lab_v, red_v):
    n, c = logits_hbm.shape
    rows_per = n // _NW
    nchunks = c // 16
    wid = lax.axis_index("s") * _NC + lax.axis_index("c")
    base = wid * rows_per
    pltpu.sync_copy(labels_hbm.at[pl.ds(base, rows_per)], lab_v)

    def do_row(j, carry):
        row = base + j
        pltpu.sync_copy(logits_hbm.at[row], row_v)

        def sumsq_step(i, acc):
            v = row_v[pl.ds(i * 16, 16)]
            return acc + v * v

        acc = lax.fori_loop(0, nchunks, sumsq_step,
                            jnp.zeros((16,), jnp.float32))
        lanes = lax.iota(jnp.int32, 16)
        for k in (1, 2, 4, 8):
            red_v[pl.ds(0, 16)] = acc
            acc = acc + plsc.load_gather(red_v, [lanes ^ k])
        inv = _rsqrt_newton(jnp.maximum(acc, 1e-24))

        lab = plsc.load_gather(lab_v, [jnp.full((16,), j, jnp.int32)])
        validv = lab != -1
        lab_safe = jnp.where(validv, lab, 0)
        t = plsc.load_gather(row_v, [lab_safe]) * inv
        t_clip = jnp.clip(t, -1.0, 1.0)
        s2 = jnp.maximum(1.0 - t_clip * t_clip, 1e-30)
        sin_theta = s2 * _rsqrt_newton(s2)
        with_margin = t_clip * _COS_M - _SIN_M * sin_theta
        new_val = jnp.where(validv, with_margin, t) * _SCALE

        scale = inv * _SCALE

        def scale_step(i, carry2):
            sl = pl.ds(i * 16, 16)
            row_v[sl] = row_v[sl] * scale
            return carry2

        lax.fori_loop(0, nchunks, scale_step, 0)

        lane0 = lanes == 0
        plsc.store_scatter(row_v, [lab_safe], new_val, mask=lane0)
        pltpu.sync_copy(row_v, out_hbm.at[row])
        return carry

    lax.fori_loop(0, rows_per, do_row, 0)


@jax.jit
def _run(logits, labels):
    n, c = logits.shape
    mesh = plsc.VectorSubcoreMesh(core_axis_name="c", subcore_axis_name="s")
    return pl.kernel(
        _sc_body,
        out_type=jax.ShapeDtypeStruct((n, c), jnp.float32),
        mesh=mesh,
        scratch_types=[
            pltpu.VMEM((c,), jnp.float32),
            pltpu.VMEM((n // _NW,), jnp.int32),
            pltpu.VMEM((128,), jnp.float32),
        ],
        compiler_params=pltpu.CompilerParams(needs_layout_passes=False),
    )(logits, labels)


def kernel(logits, labels):
    return _run(logits, labels.astype(jnp.int32))


# speedup vs baseline: 2.1949x; 2.1949x over previous
import functools
import math

import jax
import jax.numpy as jnp
from jax import lax
from jax.experimental import pallas as pl
from jax.experimental.pallas import tpu as pltpu
from jax.experimental.pallas import tpu_sc as plsc

_SCALE = 64.0
_COS_M = math.cos(0.5)
_SIN_M = math.sin(0.5)

_NC = 2
_NS = 16
_NW = _NC * _NS


def _rsqrt_newton(x):
    i = plsc.bitcast(x, jnp.int32)
    y = plsc.bitcast(jnp.int32(0x5F3759DF) - (i >> 1), jnp.float32)
    for _ in range(4):
        y = y * (1.5 - 0.5 * x * y * y)
    return y


def _sc_body(logits_hbm, labels_hbm, out_hbm, row_v, lab_v, red_v):
    n, c = logits_hbm.shape
    rows_per = n // _NW
    nchunks = c // 16
    wid = lax.axis_index("s") * _NC + lax.axis_index("c")
    base = wid * rows_per
    pltpu.sync_copy(labels_hbm.at[pl.ds(base, rows_per)], lab_v)

    def do_row(j, carry):
        row = base + j
        pltpu.sync_copy(logits_hbm.at[row], row_v)

        zero = jnp.zeros((16,), jnp.float32)

        @plsc.parallel_loop(0, c, step=160, unroll=5,
                            carry=(zero, zero, zero, zero, zero))
        def _sumsq(i, accs):
            outs = []
            for u, a in enumerate(accs):
                v0 = row_v[pl.ds(i + u * 32, 16)]
                v1 = row_v[pl.ds(i + u * 32 + 16, 16)]
                outs.append(a + v0 * v0 + v1 * v1)
            return tuple(outs)

        a0, a1, a2, a3, a4 = _sumsq
        acc = ((a0 + a1) + (a2 + a3)) + a4
        lanes = lax.iota(jnp.int32, 16)
        for k in (1, 2, 4, 8):
            red_v[pl.ds(0, 16)] = acc
            acc = acc + plsc.load_gather(red_v, [lanes ^ k])
        inv = _rsqrt_newton(jnp.maximum(acc, 1e-24))

        lab = plsc.load_gather(lab_v, [jnp.full((16,), j, jnp.int32)])
        validv = lab != -1
        lab_safe = jnp.where(validv, lab, 0)
        t = plsc.load_gather(row_v, [lab_safe]) * inv
        t_clip = jnp.clip(t, -1.0, 1.0)
        s2 = jnp.maximum(1.0 - t_clip * t_clip, 1e-30)
        sin_theta = s2 * _rsqrt_newton(s2)
        with_margin = t_clip * _COS_M - _SIN_M * sin_theta
        new_val = jnp.where(validv, with_margin, t) * _SCALE

        scale = inv * _SCALE

        @plsc.parallel_loop(0, c, step=160, unroll=5)
        def _scale(i):
            for u in range(10):
                sl = pl.ds(i + u * 16, 16)
                row_v[sl] = row_v[sl] * scale

        lane0 = lanes == 0
        plsc.store_scatter(row_v, [lab_safe], new_val, mask=lane0)
        pltpu.sync_copy(row_v, out_hbm.at[row])
        return carry

    lax.fori_loop(0, rows_per, do_row, 0)


@jax.jit
def _run(logits, labels):
    n, c = logits.shape
    mesh = plsc.VectorSubcoreMesh(core_axis_name="c", subcore_axis_name="s")
    return pl.kernel(
        _sc_body,
        out_type=jax.ShapeDtypeStruct((n, c), jnp.float32),
        mesh=mesh,
        scratch_types=[
            pltpu.VMEM((c,), jnp.float32),
            pltpu.VMEM((n // _NW,), jnp.int32),
            pltpu.VMEM((128,), jnp.float32),
        ],
        compiler_params=pltpu.CompilerParams(needs_layout_passes=False),
    )(logits, labels)


def kernel(logits, labels):
    return _run(logits, labels.astype(jnp.int32))
